# Initial kernel scaffold; baseline (speedup 1.0000x reference)
import functools

import jax
import jax.numpy as jnp
from jax import lax
from jax.experimental import pallas as pl
from jax.experimental.pallas import tpu as pltpu
from jax.experimental.pallas import tpu_sc as plsc

TILE_N = 256
TILE_MLP = 512
LANES = 16
NWORKERS = 32


def _knn_body(ct_ref, p_ref, idx_ref, w_ref, *, m_total):
    ct = ct_ref[0]
    p = p_ref[0]
    d2 = (jnp.sum(ct * ct, axis=1, keepdims=True)
          - 2.0 * jnp.dot(ct, p, preferred_element_type=jnp.float32)
          + jnp.sum(p * p, axis=0, keepdims=True))
    iota = lax.broadcasted_iota(jnp.int32, d2.shape, 0)
    idxs, invs = [], []
    d2w = d2
    for k in range(3):
        mv = jnp.min(d2w, axis=0, keepdims=True)
        mi = jnp.min(jnp.where(d2w == mv, iota, m_total), axis=0,
                     keepdims=True)
        idxs.append(mi)
        invs.append(1.0 / (jnp.sqrt(jnp.maximum(mv, 1e-10)) + 1e-8))
        if k < 2:
            d2w = jnp.where(iota == mi, jnp.float32(jnp.inf), d2w)
    s = invs[0] + invs[1] + invs[2]
    idx_ref[0] = jnp.concatenate(idxs, axis=0)
    w_ref[0] = jnp.concatenate([v / s for v in invs], axis=0)


def _mlp_body(x_ref, w1_ref, b1_ref, w2_ref, b2_ref, o_ref):
    x = x_ref[0]
    h = jnp.maximum(
        jnp.dot(w1_ref[...], x, preferred_element_type=jnp.float32)
        + b1_ref[...], 0.0)
    o_ref[0] = jnp.maximum(
        jnp.dot(w2_ref[...], h, preferred_element_type=jnp.float32)
        + b2_ref[...], 0.0)


def kernel(points_coords, centers_coords, centers_features, condition,
           W1, b1, W2, b2):
    B, _, N = points_coords.shape
    M = centers_coords.shape[2]
    CIN = centers_features.shape[1]
    C1 = W1.shape[0]
    C2 = W2.shape[0]
    NT = N // TILE_N
    PTS = (B * N) // NWORKERS
    TPW = PTS // TILE_N
    WPB = N // PTS
    GROUPS = TILE_N // LANES

    ct = jnp.swapaxes(centers_coords, 1, 2)
    knn = pl.pallas_call(
        functools.partial(_knn_body, m_total=M),
        grid=(B, NT),
        in_specs=[
            pl.BlockSpec((1, M, 3), lambda b, t: (b, 0, 0)),
            pl.BlockSpec((1, 3, TILE_N), lambda b, t: (b, 0, t)),
        ],
        out_specs=[
            pl.BlockSpec((1, 3, TILE_N), lambda b, t: (b * NT + t, 0, 0)),
            pl.BlockSpec((1, 3, TILE_N), lambda b, t: (b * NT + t, 0, 0)),
        ],
        out_shape=[
            jax.ShapeDtypeStruct((B * NT, 3, TILE_N), jnp.int32),
            jax.ShapeDtypeStruct((B * NT, 3, TILE_N), jnp.float32),
        ],
    )
    idx, w = knn(ct, points_coords)

    mesh = plsc.VectorSubcoreMesh(core_axis_name="c", subcore_axis_name="s")

    @functools.partial(
        pl.kernel, mesh=mesh,
        out_type=jax.ShapeDtypeStruct((B, CIN, N), jnp.float32),
        scratch_types=[
            pltpu.VMEM((CIN, M), jnp.float32),
            pltpu.VMEM((TPW, 3, TILE_N), jnp.int32),
            pltpu.VMEM((TPW, 3, TILE_N), jnp.float32),
            pltpu.VMEM((CIN, PTS), jnp.float32),
        ],
    )
    def sc_interp(feat_hbm, idx_hbm, w_hbm, out_hbm,
                  table_v, idx_v, w_v, out_v):
        wid = lax.axis_index("s") * 2 + lax.axis_index("c")
        b = wid // WPB
        chunk = wid % WPB
        pltpu.sync_copy(feat_hbm.at[b], table_v)
        pltpu.sync_copy(idx_hbm.at[pl.ds(wid * TPW, TPW)], idx_v)
        pltpu.sync_copy(w_hbm.at[pl.ds(wid * TPW, TPW)], w_v)
        for t in range(TPW):
            def body(g, carry, t=t):
                i0 = idx_v[t, 0, pl.ds(g * LANES, LANES)]
                i1 = idx_v[t, 1, pl.ds(g * LANES, LANES)]
                i2 = idx_v[t, 2, pl.ds(g * LANES, LANES)]
                w0 = w_v[t, 0, pl.ds(g * LANES, LANES)]
                w1_ = w_v[t, 1, pl.ds(g * LANES, LANES)]
                w2_ = w_v[t, 2, pl.ds(g * LANES, LANES)]
                off = t * TILE_N + g * LANES
                for c in range(CIN):
                    row = table_v.at[c]
                    acc = (w0 * plsc.load_gather(row, [i0])
                           + w1_ * plsc.load_gather(row, [i1])
                           + w2_ * plsc.load_gather(row, [i2]))
                    out_v[c, pl.ds(off, LANES)] = acc
                return carry
            lax.fori_loop(0, GROUPS, body, 0)
        pltpu.sync_copy(out_v, out_hbm.at[b, :, pl.ds(chunk * PTS, PTS)])

    interp = sc_interp(centers_features, idx, w)

    mlp = pl.pallas_call(
        _mlp_body,
        grid=(B, N // TILE_MLP),
        in_specs=[
            pl.BlockSpec((1, CIN, TILE_MLP), lambda b, t: (b, 0, t)),
            pl.BlockSpec((C1, CIN), lambda b, t: (0, 0)),
            pl.BlockSpec((C1, 1), lambda b, t: (0, 0)),
            pl.BlockSpec((C2, C1), lambda b, t: (0, 0)),
            pl.BlockSpec((C2, 1), lambda b, t: (0, 0)),
        ],
        out_specs=pl.BlockSpec((1, C2, TILE_MLP), lambda b, t: (b, 0, t)),
        out_shape=jax.ShapeDtypeStruct((B, C2, N), jnp.float32),
    )
    out = mlp(interp, W1, b1.reshape(C1, 1), W2, b2.reshape(C2, 1))
    return (out, points_coords, condition)


# speedup vs baseline: 27.7155x; 27.7155x over previous
import functools

import jax
import jax.numpy as jnp
from jax import lax
from jax.experimental import pallas as pl
from jax.experimental.pallas import tpu as pltpu
from jax.experimental.pallas import tpu_sc as plsc

TILE_N = 256
TILE_MLP = 512
LANES = 16
NWORKERS = 32


def _knn_body(ct_ref, p_ref, idx_ref, w_ref, *, m_total):
    ct = ct_ref[0]
    p = p_ref[0]
    d2 = (jnp.sum(ct * ct, axis=1, keepdims=True)
          - 2.0 * jnp.dot(ct, p, preferred_element_type=jnp.float32)
          + jnp.sum(p * p, axis=0, keepdims=True))
    iota = lax.broadcasted_iota(jnp.int32, d2.shape, 0)
    idxs, invs = [], []
    d2w = d2
    for k in range(3):
        mv = jnp.min(d2w, axis=0, keepdims=True)
        mi = jnp.min(jnp.where(d2w == mv, iota, m_total), axis=0,
                     keepdims=True)
        idxs.append(mi)
        invs.append(1.0 / (jnp.sqrt(jnp.maximum(mv, 1e-10)) + 1e-8))
        if k < 2:
            d2w = jnp.where(iota == mi, jnp.float32(jnp.inf), d2w)
    s = invs[0] + invs[1] + invs[2]
    idx_ref[0] = jnp.concatenate(idxs, axis=0)
    w_ref[0] = jnp.concatenate([v / s for v in invs], axis=0)


def _mlp_body(x_ref, w1_ref, b1_ref, w2_ref, b2_ref, o_ref):
    x = x_ref[0]
    h = jnp.maximum(
        jnp.dot(w1_ref[...], x, preferred_element_type=jnp.float32)
        + b1_ref[...], 0.0)
    o_ref[0] = jnp.maximum(
        jnp.dot(w2_ref[...], h, preferred_element_type=jnp.float32)
        + b2_ref[...], 0.0)


def kernel(points_coords, centers_coords, centers_features, condition,
           W1, b1, W2, b2):
    B, _, N = points_coords.shape
    M = centers_coords.shape[2]
    CIN = centers_features.shape[1]
    C1 = W1.shape[0]
    C2 = W2.shape[0]
    NT = N // TILE_N
    PTS = (B * N) // NWORKERS
    TPW = PTS // TILE_N
    WPB = N // PTS
    GROUPS = TILE_N // LANES

    ct = jnp.swapaxes(centers_coords, 1, 2)
    knn = pl.pallas_call(
        functools.partial(_knn_body, m_total=M),
        grid=(B, NT),
        in_specs=[
            pl.BlockSpec((1, M, 3), lambda b, t: (b, 0, 0)),
            pl.BlockSpec((1, 3, TILE_N), lambda b, t: (b, 0, t)),
        ],
        out_specs=[
            pl.BlockSpec((1, 3, TILE_N), lambda b, t: (b * NT + t, 0, 0)),
            pl.BlockSpec((1, 3, TILE_N), lambda b, t: (b * NT + t, 0, 0)),
        ],
        out_shape=[
            jax.ShapeDtypeStruct((B * NT, 3, TILE_N), jnp.int32),
            jax.ShapeDtypeStruct((B * NT, 3, TILE_N), jnp.float32),
        ],
    )
    idx, w = knn(ct, points_coords)

    mesh = plsc.VectorSubcoreMesh(core_axis_name="c", subcore_axis_name="s")

    @functools.partial(
        pl.kernel, mesh=mesh,
        compiler_params=pltpu.CompilerParams(
            use_tc_tiling_on_sc=False, needs_layout_passes=False),
        out_type=jax.ShapeDtypeStruct((B, CIN, N), jnp.float32),
        scratch_types=[
            pltpu.VMEM((CIN, M), jnp.float32),
            pltpu.VMEM((TPW, 3, TILE_N), jnp.int32),
            pltpu.VMEM((TPW, 3, TILE_N), jnp.float32),
            pltpu.VMEM((CIN, PTS), jnp.float32),
        ],
    )
    def sc_interp(feat_hbm, idx_hbm, w_hbm, out_hbm,
                  table_v, idx_v, w_v, out_v):
        wid = lax.axis_index("s") * 2 + lax.axis_index("c")
        b = wid // WPB
        chunk = wid % WPB
        pltpu.sync_copy(feat_hbm.at[b], table_v)
        pltpu.sync_copy(idx_hbm.at[pl.ds(wid * TPW, TPW)], idx_v)
        pltpu.sync_copy(w_hbm.at[pl.ds(wid * TPW, TPW)], w_v)
        for t in range(TPW):
            def body(g, carry, t=t):
                i0 = idx_v[t, 0, pl.ds(g * LANES, LANES)]
                i1 = idx_v[t, 1, pl.ds(g * LANES, LANES)]
                i2 = idx_v[t, 2, pl.ds(g * LANES, LANES)]
                w0 = w_v[t, 0, pl.ds(g * LANES, LANES)]
                w1_ = w_v[t, 1, pl.ds(g * LANES, LANES)]
                w2_ = w_v[t, 2, pl.ds(g * LANES, LANES)]
                off = t * TILE_N + g * LANES
                for c in range(CIN):
                    cvec = jnp.full((LANES,), c, jnp.int32)
                    acc = (w0 * plsc.load_gather(table_v, [cvec, i0])
                           + w1_ * plsc.load_gather(table_v, [cvec, i1])
                           + w2_ * plsc.load_gather(table_v, [cvec, i2]))
                    out_v[c, pl.ds(off, LANES)] = acc
                return carry
            lax.fori_loop(0, GROUPS, body, 0)
        pltpu.sync_copy(out_v, out_hbm.at[b, :, pl.ds(chunk * PTS, PTS)])

    interp = sc_interp(centers_features, idx, w)

    mlp = pl.pallas_call(
        _mlp_body,
        grid=(B, N // TILE_MLP),
        in_specs=[
            pl.BlockSpec((1, CIN, TILE_MLP), lambda b, t: (b, 0, t)),
            pl.BlockSpec((C1, CIN), lambda b, t: (0, 0)),
            pl.BlockSpec((C1, 1), lambda b, t: (0, 0)),
            pl.BlockSpec((C2, C1), lambda b, t: (0, 0)),
            pl.BlockSpec((C2, 1), lambda b, t: (0, 0)),
        ],
        out_specs=pl.BlockSpec((1, C2, TILE_MLP), lambda b, t: (b, 0, t)),
        out_shape=jax.ShapeDtypeStruct((B, C2, N), jnp.float32),
    )
    out = mlp(interp, W1, b1.reshape(C1, 1), W2, b2.reshape(C2, 1))
    return (out, points_coords, condition)


# speedup vs baseline: 30.0205x; 1.0832x over previous
import functools

import jax
import jax.numpy as jnp
from jax import lax
from jax.experimental import pallas as pl
from jax.experimental.pallas import tpu as pltpu
from jax.experimental.pallas import tpu_sc as plsc

TILE_N = 512
TILE_MLP = 512
LANES = 16
NWORKERS = 32


def _knn_body(ct_ref, p_ref, idx_ref, w_ref, *, m_total):
    ct = ct_ref[0]
    p = p_ref[0]
    pn2 = jnp.sum(p * p, axis=0, keepdims=True)
    cm2 = jnp.sum(ct * ct, axis=1, keepdims=True)
    aug = jnp.concatenate([ct * -2.0, cm2, jnp.ones_like(cm2)], axis=1)
    p_aug = jnp.concatenate([p, jnp.ones_like(pn2), pn2], axis=0)
    d2 = jnp.dot(aug, p_aug, preferred_element_type=jnp.float32)
    iotaf = lax.broadcasted_iota(jnp.int32, d2.shape, 0).astype(jnp.float32)
    idxs, invs = [], []
    d2w = d2
    for k in range(3):
        mv = jnp.min(d2w, axis=0, keepdims=True)
        mi = jnp.min(jnp.where(d2w == mv, iotaf, jnp.float32(m_total)),
                     axis=0, keepdims=True)
        idxs.append(mi.astype(jnp.int32))
        invs.append(1.0 / (jnp.sqrt(jnp.maximum(mv, 1e-10)) + 1e-8))
        if k < 2:
            d2w = jnp.where(iotaf == mi, jnp.float32(jnp.inf), d2w)
    s = invs[0] + invs[1] + invs[2]
    idx_ref[0] = jnp.concatenate(idxs, axis=0)
    w_ref[0] = jnp.concatenate([v / s for v in invs], axis=0)


def _mlp_body(x_ref, w1_ref, b1_ref, w2_ref, b2_ref, o_ref):
    x = x_ref[0]
    h = jnp.maximum(
        jnp.dot(w1_ref[...], x, preferred_element_type=jnp.float32)
        + b1_ref[...], 0.0)
    o_ref[0] = jnp.maximum(
        jnp.dot(w2_ref[...], h, preferred_element_type=jnp.float32)
        + b2_ref[...], 0.0)


def kernel(points_coords, centers_coords, centers_features, condition,
           W1, b1, W2, b2):
    B, _, N = points_coords.shape
    M = centers_coords.shape[2]
    CIN = centers_features.shape[1]
    C1 = W1.shape[0]
    C2 = W2.shape[0]
    NT = N // TILE_N
    PTS = (B * N) // NWORKERS
    TPW = PTS // TILE_N
    WPB = N // PTS
    GROUPS = TILE_N // LANES

    ct = jnp.swapaxes(centers_coords, 1, 2)
    knn = pl.pallas_call(
        functools.partial(_knn_body, m_total=M),
        grid=(B, NT),
        in_specs=[
            pl.BlockSpec((1, M, 3), lambda b, t: (b, 0, 0)),
            pl.BlockSpec((1, 3, TILE_N), lambda b, t: (b, 0, t)),
        ],
        out_specs=[
            pl.BlockSpec((1, 3, TILE_N), lambda b, t: (b * NT + t, 0, 0)),
            pl.BlockSpec((1, 3, TILE_N), lambda b, t: (b * NT + t, 0, 0)),
        ],
        out_shape=[
            jax.ShapeDtypeStruct((B * NT, 3, TILE_N), jnp.int32),
            jax.ShapeDtypeStruct((B * NT, 3, TILE_N), jnp.float32),
        ],
    )
    idx, w = knn(ct, points_coords)

    mesh = plsc.VectorSubcoreMesh(core_axis_name="c", subcore_axis_name="s")

    @functools.partial(
        pl.kernel, mesh=mesh,
        compiler_params=pltpu.CompilerParams(
            use_tc_tiling_on_sc=False, needs_layout_passes=False),
        out_type=jax.ShapeDtypeStruct((B, CIN, N), jnp.float32),
        scratch_types=[
            pltpu.VMEM((CIN, M), jnp.float32),
            pltpu.VMEM((TPW, 3, TILE_N), jnp.int32),
            pltpu.VMEM((TPW, 3, TILE_N), jnp.float32),
            pltpu.VMEM((CIN, PTS), jnp.float32),
        ],
    )
    def sc_interp(feat_hbm, idx_hbm, w_hbm, out_hbm,
                  table_v, idx_v, w_v, out_v):
        wid = lax.axis_index("s") * 2 + lax.axis_index("c")
        b = wid // WPB
        chunk = wid % WPB
        pltpu.sync_copy(feat_hbm.at[b], table_v)
        pltpu.sync_copy(idx_hbm.at[pl.ds(wid * TPW, TPW)], idx_v)
        pltpu.sync_copy(w_hbm.at[pl.ds(wid * TPW, TPW)], w_v)
        for t in range(TPW):
            def body(g, carry, t=t):
                i0 = idx_v[t, 0, pl.ds(g * LANES, LANES)]
                i1 = idx_v[t, 1, pl.ds(g * LANES, LANES)]
                i2 = idx_v[t, 2, pl.ds(g * LANES, LANES)]
                w0 = w_v[t, 0, pl.ds(g * LANES, LANES)]
                w1_ = w_v[t, 1, pl.ds(g * LANES, LANES)]
                w2_ = w_v[t, 2, pl.ds(g * LANES, LANES)]
                off = t * TILE_N + g * LANES
                for c in range(CIN):
                    cvec = jnp.full((LANES,), c, jnp.int32)
                    acc = (w0 * plsc.load_gather(table_v, [cvec, i0])
                           + w1_ * plsc.load_gather(table_v, [cvec, i1])
                           + w2_ * plsc.load_gather(table_v, [cvec, i2]))
                    out_v[c, pl.ds(off, LANES)] = acc
                return carry
            lax.fori_loop(0, GROUPS, body, 0)
        pltpu.sync_copy(out_v, out_hbm.at[b, :, pl.ds(chunk * PTS, PTS)])

    interp = sc_interp(centers_features, idx, w)

    mlp = pl.pallas_call(
        _mlp_body,
        grid=(B, N // TILE_MLP),
        in_specs=[
            pl.BlockSpec((1, CIN, TILE_MLP), lambda b, t: (b, 0, t)),
            pl.BlockSpec((C1, CIN), lambda b, t: (0, 0)),
            pl.BlockSpec((C1, 1), lambda b, t: (0, 0)),
            pl.BlockSpec((C2, C1), lambda b, t: (0, 0)),
            pl.BlockSpec((C2, 1), lambda b, t: (0, 0)),
        ],
        out_specs=pl.BlockSpec((1, C2, TILE_MLP), lambda b, t: (b, 0, t)),
        out_shape=jax.ShapeDtypeStruct((B, C2, N), jnp.float32),
    )
    out = mlp(interp, W1, b1.reshape(C1, 1), W2, b2.reshape(C2, 1))
    return (out, points_coords, condition)


# speedup vs baseline: 33.7148x; 1.1231x over previous
import functools

import jax
import jax.numpy as jnp
from jax import lax
from jax.experimental import pallas as pl
from jax.experimental.pallas import tpu as pltpu
from jax.experimental.pallas import tpu_sc as plsc

TILE_N = 512
TILE_MLP = 512
LANES = 16
NWORKERS = 32


def _knn_body(ct_ref, p_ref, idx_ref, w_ref, *, m_total):
    ct = ct_ref[0]
    p = p_ref[0]
    d2 = (jnp.sum(ct * ct, axis=1, keepdims=True)
          - 2.0 * jnp.dot(ct, p, preferred_element_type=jnp.float32)
          + jnp.sum(p * p, axis=0, keepdims=True))
    iotaf = lax.broadcasted_iota(jnp.int32, d2.shape, 0).astype(jnp.float32)
    idxs, invs = [], []
    d2w = d2
    for k in range(3):
        mv = jnp.min(d2w, axis=0, keepdims=True)
        mi = jnp.min(jnp.where(d2w == mv, iotaf, jnp.float32(m_total)),
                     axis=0, keepdims=True)
        idxs.append(mi.astype(jnp.int32))
        invs.append(1.0 / (jnp.sqrt(jnp.maximum(mv, 1e-10)) + 1e-8))
        if k < 2:
            d2w = jnp.where(iotaf == mi, jnp.float32(jnp.inf), d2w)
    s = invs[0] + invs[1] + invs[2]
    idx_ref[0] = jnp.concatenate(idxs, axis=0)
    w_ref[0] = jnp.concatenate([v / s for v in invs], axis=0)


def _mlp_body(x_ref, w1_ref, b1_ref, w2_ref, b2_ref, o_ref):
    x = x_ref[0]
    h = jnp.maximum(
        jnp.dot(w1_ref[...], x, preferred_element_type=jnp.float32)
        + b1_ref[...], 0.0)
    o_ref[0] = jnp.maximum(
        jnp.dot(w2_ref[...], h, preferred_element_type=jnp.float32)
        + b2_ref[...], 0.0)


def kernel(points_coords, centers_coords, centers_features, condition,
           W1, b1, W2, b2):
    B, _, N = points_coords.shape
    M = centers_coords.shape[2]
    CIN = centers_features.shape[1]
    C1 = W1.shape[0]
    C2 = W2.shape[0]
    NT = N // TILE_N
    PTS = (B * N) // NWORKERS
    TPW = PTS // TILE_N
    WPB = N // PTS
    GROUPS = TILE_N // LANES

    ct = jnp.swapaxes(centers_coords, 1, 2)
    knn = pl.pallas_call(
        functools.partial(_knn_body, m_total=M),
        grid=(B, NT),
        in_specs=[
            pl.BlockSpec((1, M, 3), lambda b, t: (b, 0, 0)),
            pl.BlockSpec((1, 3, TILE_N), lambda b, t: (b, 0, t)),
        ],
        out_specs=[
            pl.BlockSpec((1, 3, TILE_N), lambda b, t: (b * NT + t, 0, 0)),
            pl.BlockSpec((1, 3, TILE_N), lambda b, t: (b * NT + t, 0, 0)),
        ],
        out_shape=[
            jax.ShapeDtypeStruct((B * NT, 3, TILE_N), jnp.int32),
            jax.ShapeDtypeStruct((B * NT, 3, TILE_N), jnp.float32),
        ],
    )
    idx, w = knn(ct, points_coords)

    mesh = plsc.VectorSubcoreMesh(core_axis_name="c", subcore_axis_name="s")

    @functools.partial(
        pl.kernel, mesh=mesh,
        compiler_params=pltpu.CompilerParams(
            use_tc_tiling_on_sc=False, needs_layout_passes=False),
        out_type=jax.ShapeDtypeStruct((B, CIN, N), jnp.float32),
        scratch_types=[
            pltpu.VMEM((CIN, M), jnp.float32),
            pltpu.VMEM((TPW, 3, TILE_N), jnp.int32),
            pltpu.VMEM((TPW, 3, TILE_N), jnp.float32),
            pltpu.VMEM((CIN, PTS), jnp.float32),
        ],
    )
    def sc_interp(feat_hbm, idx_hbm, w_hbm, out_hbm,
                  table_v, idx_v, w_v, out_v):
        wid = lax.axis_index("s") * 2 + lax.axis_index("c")
        b = wid // WPB
        chunk = wid % WPB
        pltpu.sync_copy(feat_hbm.at[b], table_v)
        pltpu.sync_copy(idx_hbm.at[pl.ds(wid * TPW, TPW)], idx_v)
        pltpu.sync_copy(w_hbm.at[pl.ds(wid * TPW, TPW)], w_v)
        for t in range(TPW):
            def body(g, carry, t=t):
                i0 = idx_v[t, 0, pl.ds(g * LANES, LANES)]
                i1 = idx_v[t, 1, pl.ds(g * LANES, LANES)]
                i2 = idx_v[t, 2, pl.ds(g * LANES, LANES)]
                w0 = w_v[t, 0, pl.ds(g * LANES, LANES)]
                w1_ = w_v[t, 1, pl.ds(g * LANES, LANES)]
                w2_ = w_v[t, 2, pl.ds(g * LANES, LANES)]
                off = t * TILE_N + g * LANES
                for c in range(CIN):
                    cvec = jnp.full((LANES,), c, jnp.int32)
                    acc = (w0 * plsc.load_gather(table_v, [cvec, i0])
                           + w1_ * plsc.load_gather(table_v, [cvec, i1])
                           + w2_ * plsc.load_gather(table_v, [cvec, i2]))
                    out_v[c, pl.ds(off, LANES)] = acc
                return carry
            lax.fori_loop(0, GROUPS, body, 0)
        pltpu.sync_copy(out_v, out_hbm.at[b, :, pl.ds(chunk * PTS, PTS)])

    interp = sc_interp(centers_features, idx, w)

    mlp = pl.pallas_call(
        _mlp_body,
        grid=(B, N // TILE_MLP),
        in_specs=[
            pl.BlockSpec((1, CIN, TILE_MLP), lambda b, t: (b, 0, t)),
            pl.BlockSpec((C1, CIN), lambda b, t: (0, 0)),
            pl.BlockSpec((C1, 1), lambda b, t: (0, 0)),
            pl.BlockSpec((C2, C1), lambda b, t: (0, 0)),
            pl.BlockSpec((C2, 1), lambda b, t: (0, 0)),
        ],
        out_specs=pl.BlockSpec((1, C2, TILE_MLP), lambda b, t: (b, 0, t)),
        out_shape=jax.ShapeDtypeStruct((B, C2, N), jnp.float32),
    )
    out = mlp(interp, W1, b1.reshape(C1, 1), W2, b2.reshape(C2, 1))
    return (out, points_coords, condition)


# speedup vs baseline: 38.1391x; 1.1312x over previous
import functools

import jax
import jax.numpy as jnp
from jax import lax
from jax.experimental import pallas as pl
from jax.experimental.pallas import tpu as pltpu
from jax.experimental.pallas import tpu_sc as plsc

TILE_N = 512
TILE_MLP = 1024
LANES = 16
NWORKERS = 32


def _knn_body(ct_ref, p_ref, idx_ref, w_ref, *, m_total):
    ct = ct_ref[...]
    p = p_ref[...]
    d2 = (jnp.sum(ct * ct, axis=1, keepdims=True)
          - 2.0 * jnp.dot(ct, p, preferred_element_type=jnp.float32)
          + jnp.sum(p * p, axis=0, keepdims=True))
    iotaf = lax.broadcasted_iota(jnp.int32, d2.shape, 0).astype(jnp.float32)
    idxs, invs = [], []
    d2w = d2
    for k in range(3):
        mv = jnp.min(d2w, axis=0, keepdims=True)
        mi = jnp.min(jnp.where(d2w == mv, iotaf, jnp.float32(m_total)),
                     axis=0, keepdims=True)
        idxs.append(mi.astype(jnp.int32))
        invs.append(1.0 / (jnp.sqrt(jnp.maximum(mv, 1e-10)) + 1e-8))
        if k < 2:
            d2w = jnp.where(iotaf == mi, jnp.float32(jnp.inf), d2w)
    s = invs[0] + invs[1] + invs[2]
    idx_ref[0] = jnp.concatenate(idxs, axis=0)
    w_ref[0] = jnp.concatenate([v / s for v in invs], axis=0)


def _mlp_body(x_ref, w1_ref, b1_ref, w2_ref, b2_ref, o_ref):
    h = jnp.maximum(
        jnp.dot(w1_ref[...], x_ref[...], preferred_element_type=jnp.float32)
        + b1_ref[...], 0.0)
    o_ref[...] = jnp.maximum(
        jnp.dot(w2_ref[...], h, preferred_element_type=jnp.float32)
        + b2_ref[...], 0.0)


def kernel(points_coords, centers_coords, centers_features, condition,
           W1, b1, W2, b2):
    B, _, N = points_coords.shape
    M = centers_coords.shape[2]
    CIN = centers_features.shape[1]
    C1 = W1.shape[0]
    C2 = W2.shape[0]
    NT = N // TILE_N
    PTS = N // NWORKERS
    TPW = PTS // TILE_N
    GROUPS = PTS // LANES

    knn = pl.pallas_call(
        functools.partial(_knn_body, m_total=M),
        grid=(NT,),
        in_specs=[
            pl.BlockSpec((M, 3), lambda t: (0, 0)),
            pl.BlockSpec((3, TILE_N), lambda t: (0, t)),
        ],
        out_specs=[
            pl.BlockSpec((1, 3, TILE_N), lambda t: (t, 0, 0)),
            pl.BlockSpec((1, 3, TILE_N), lambda t: (t, 0, 0)),
        ],
        out_shape=[
            jax.ShapeDtypeStruct((NT, 3, TILE_N), jnp.int32),
            jax.ShapeDtypeStruct((NT, 3, TILE_N), jnp.float32),
        ],
    )

    mesh = plsc.VectorSubcoreMesh(core_axis_name="c", subcore_axis_name="s")

    @functools.partial(
        pl.kernel, mesh=mesh,
        compiler_params=pltpu.CompilerParams(
            use_tc_tiling_on_sc=False, needs_layout_passes=False),
        out_type=jax.ShapeDtypeStruct((CIN, N), jnp.float32),
        scratch_types=[
            pltpu.VMEM((CIN, M), jnp.float32),
            pltpu.VMEM((TPW, 3, TILE_N), jnp.int32),
            pltpu.VMEM((TPW, 3, TILE_N), jnp.float32),
            pltpu.VMEM((CIN, PTS), jnp.float32),
        ],
    )
    def sc_interp(feat_hbm, idx_hbm, w_hbm, out_hbm,
                  table_v, idx_v, w_v, out_v):
        wid = lax.axis_index("s") * 2 + lax.axis_index("c")
        pltpu.sync_copy(feat_hbm, table_v)
        pltpu.sync_copy(idx_hbm.at[pl.ds(wid * TPW, TPW)], idx_v)
        pltpu.sync_copy(w_hbm.at[pl.ds(wid * TPW, TPW)], w_v)
        for t in range(TPW):
            def body(g, carry, t=t):
                i0 = idx_v[t, 0, pl.ds(g * LANES, LANES)]
                i1 = idx_v[t, 1, pl.ds(g * LANES, LANES)]
                i2 = idx_v[t, 2, pl.ds(g * LANES, LANES)]
                w0 = w_v[t, 0, pl.ds(g * LANES, LANES)]
                w1_ = w_v[t, 1, pl.ds(g * LANES, LANES)]
                w2_ = w_v[t, 2, pl.ds(g * LANES, LANES)]
                off = t * TILE_N + g * LANES
                for c in range(CIN):
                    cvec = jnp.full((LANES,), c, jnp.int32)
                    acc = (w0 * plsc.load_gather(table_v, [cvec, i0])
                           + w1_ * plsc.load_gather(table_v, [cvec, i1])
                           + w2_ * plsc.load_gather(table_v, [cvec, i2]))
                    out_v[c, pl.ds(off, LANES)] = acc
                return carry
            lax.fori_loop(0, TILE_N // LANES, body, 0)
        pltpu.sync_copy(out_v, out_hbm.at[:, pl.ds(wid * PTS, PTS)])

    mlp = pl.pallas_call(
        _mlp_body,
        grid=(N // TILE_MLP,),
        in_specs=[
            pl.BlockSpec((CIN, TILE_MLP), lambda t: (0, t)),
            pl.BlockSpec((C1, CIN), lambda t: (0, 0)),
            pl.BlockSpec((C1, 1), lambda t: (0, 0)),
            pl.BlockSpec((C2, C1), lambda t: (0, 0)),
            pl.BlockSpec((C2, 1), lambda t: (0, 0)),
        ],
        out_specs=pl.BlockSpec((C2, TILE_MLP), lambda t: (0, t)),
        out_shape=jax.ShapeDtypeStruct((C2, N), jnp.float32),
    )

    ct = jnp.swapaxes(centers_coords, 1, 2)
    b1c = b1.reshape(C1, 1)
    b2c = b2.reshape(C2, 1)
    outs = []
    for b in range(B):
        idx, w = knn(ct[b], points_coords[b])
        interp = sc_interp(centers_features[b], idx, w)
        outs.append(mlp(interp, W1, b1c, W2, b2c))
    return (jnp.stack(outs), points_coords, condition)


# speedup vs baseline: 38.1580x; 1.0005x over previous
import functools

import jax
import jax.numpy as jnp
from jax import lax
from jax.experimental import pallas as pl
from jax.experimental.pallas import tpu as pltpu
from jax.experimental.pallas import tpu_sc as plsc

TILE_N = 512
TILE_MLP = 1024
LANES = 16
NWORKERS = 32


def _knn_body(ct_ref, p_ref, idx_ref, w_ref, *, m_total):
    ct = ct_ref[...]
    p = p_ref[...]
    d2 = (jnp.sum(ct * ct, axis=1, keepdims=True)
          - 2.0 * jnp.dot(ct, p, preferred_element_type=jnp.float32)
          + jnp.sum(p * p, axis=0, keepdims=True))
    iotaf = lax.broadcasted_iota(jnp.int32, d2.shape, 0).astype(jnp.float32)
    idxs, invs = [], []
    d2w = d2
    for k in range(3):
        mv = jnp.min(d2w, axis=0, keepdims=True)
        mi = jnp.min(jnp.where(d2w == mv, iotaf, jnp.float32(m_total)),
                     axis=0, keepdims=True)
        idxs.append(mi.astype(jnp.int32))
        invs.append(1.0 / (jnp.sqrt(jnp.maximum(mv, 1e-10)) + 1e-8))
        if k < 2:
            d2w = jnp.where(iotaf == mi, jnp.float32(jnp.inf), d2w)
    s = invs[0] + invs[1] + invs[2]
    idx_ref[0] = jnp.concatenate(idxs, axis=0)
    w_ref[0] = jnp.concatenate([v / s for v in invs], axis=0)


def _mlp_body(x_ref, w1_ref, b1_ref, w2_ref, b2_ref, o_ref):
    h = jnp.maximum(
        jnp.dot(w1_ref[...], x_ref[...], preferred_element_type=jnp.float32)
        + b1_ref[...], 0.0)
    o_ref[...] = jnp.maximum(
        jnp.dot(w2_ref[...], h, preferred_element_type=jnp.float32)
        + b2_ref[...], 0.0)


def kernel(points_coords, centers_coords, centers_features, condition,
           W1, b1, W2, b2):
    B, _, N = points_coords.shape
    M = centers_coords.shape[2]
    CIN = centers_features.shape[1]
    C1 = W1.shape[0]
    C2 = W2.shape[0]
    NT = N // TILE_N
    PTS = N // NWORKERS
    TPW = PTS // TILE_N
    GROUPS = PTS // LANES

    knn = pl.pallas_call(
        functools.partial(_knn_body, m_total=M),
        grid=(NT,),
        in_specs=[
            pl.BlockSpec((M, 3), lambda t: (0, 0)),
            pl.BlockSpec((3, TILE_N), lambda t: (0, t)),
        ],
        out_specs=[
            pl.BlockSpec((1, 3, TILE_N), lambda t: (t, 0, 0)),
            pl.BlockSpec((1, 3, TILE_N), lambda t: (t, 0, 0)),
        ],
        out_shape=[
            jax.ShapeDtypeStruct((NT, 3, TILE_N), jnp.int32),
            jax.ShapeDtypeStruct((NT, 3, TILE_N), jnp.float32),
        ],
    )

    mesh = plsc.VectorSubcoreMesh(core_axis_name="c", subcore_axis_name="s")

    @functools.partial(
        pl.kernel, mesh=mesh,
        compiler_params=pltpu.CompilerParams(
            use_tc_tiling_on_sc=False, needs_layout_passes=False),
        out_type=jax.ShapeDtypeStruct((CIN, N), jnp.float32),
        scratch_types=[
            pltpu.VMEM((CIN, M), jnp.float32),
            pltpu.VMEM((TPW, 3, TILE_N), jnp.int32),
            pltpu.VMEM((TPW, 3, TILE_N), jnp.float32),
            pltpu.VMEM((CIN, PTS), jnp.float32),
        ],
    )
    def sc_interp(feat_hbm, idx_hbm, w_hbm, out_hbm,
                  table_v, idx_v, w_v, out_v):
        wid = lax.axis_index("s") * 2 + lax.axis_index("c")
        pltpu.sync_copy(feat_hbm, table_v)
        pltpu.sync_copy(idx_hbm.at[pl.ds(wid * TPW, TPW)], idx_v)
        pltpu.sync_copy(w_hbm.at[pl.ds(wid * TPW, TPW)], w_v)
        for t in range(TPW):
            @plsc.parallel_loop(0, TILE_N // LANES, unroll=4)
            def body(g, t=t):
                i0 = idx_v[t, 0, pl.ds(g * LANES, LANES)]
                i1 = idx_v[t, 1, pl.ds(g * LANES, LANES)]
                i2 = idx_v[t, 2, pl.ds(g * LANES, LANES)]
                w0 = w_v[t, 0, pl.ds(g * LANES, LANES)]
                w1_ = w_v[t, 1, pl.ds(g * LANES, LANES)]
                w2_ = w_v[t, 2, pl.ds(g * LANES, LANES)]
                off = t * TILE_N + g * LANES
                for c in range(CIN):
                    cvec = jnp.full((LANES,), c, jnp.int32)
                    acc = (w0 * plsc.load_gather(table_v, [cvec, i0])
                           + w1_ * plsc.load_gather(table_v, [cvec, i1])
                           + w2_ * plsc.load_gather(table_v, [cvec, i2]))
                    out_v[c, pl.ds(off, LANES)] = acc
        pltpu.sync_copy(out_v, out_hbm.at[:, pl.ds(wid * PTS, PTS)])

    mlp = pl.pallas_call(
        _mlp_body,
        grid=(N // TILE_MLP,),
        in_specs=[
            pl.BlockSpec((CIN, TILE_MLP), lambda t: (0, t)),
            pl.BlockSpec((C1, CIN), lambda t: (0, 0)),
            pl.BlockSpec((C1, 1), lambda t: (0, 0)),
            pl.BlockSpec((C2, C1), lambda t: (0, 0)),
            pl.BlockSpec((C2, 1), lambda t: (0, 0)),
        ],
        out_specs=pl.BlockSpec((C2, TILE_MLP), lambda t: (0, t)),
        out_shape=jax.ShapeDtypeStruct((C2, N), jnp.float32),
    )

    ct = jnp.swapaxes(centers_coords, 1, 2)
    b1c = b1.reshape(C1, 1)
    b2c = b2.reshape(C2, 1)
    res = [knn(ct[b], points_coords[b]) for b in range(B)]
    interps = [sc_interp(centers_features[b], res[b][0], res[b][1])
               for b in range(B)]
    outs = [mlp(x, W1, b1c, W2, b2c) for x in interps]
    return (jnp.stack(outs), points_coords, condition)


# speedup vs baseline: 46.3462x; 1.2146x over previous
import functools

import jax
import jax.numpy as jnp
from jax import lax
from jax.experimental import pallas as pl
from jax.experimental.pallas import tpu as pltpu
from jax.experimental.pallas import tpu_sc as plsc

TILE_N = 512
TILE_MLP = 1024
LANES = 16
NWORKERS = 32


def _knn_body(ct_ref, p_ref, idx_ref, w_ref, *, m_total):
    ct = ct_ref[...]
    p = p_ref[...]
    pn2 = jnp.sum(p * p, axis=0, keepdims=True)
    cm2 = jnp.sum(ct * ct, axis=1, keepdims=True)
    q = (cm2 - 2.0 * jnp.dot(ct, p, preferred_element_type=jnp.float32)
         + pn2)
    ii = lax.broadcasted_iota(jnp.int32, (8, q.shape[0]), 1)
    rr = lax.broadcasted_iota(jnp.int32, (8, q.shape[0]), 0)
    hi = jnp.where(rr == 0, ii >> 5, 0).astype(jnp.float32)
    lo = jnp.where(rr == 1, ii & 31, 0).astype(jnp.float32)
    iota_rows = hi + lo
    mis, invs = [], []
    qw = q
    for k in range(3):
        mv = jnp.min(qw, axis=0, keepdims=True)
        mask = qw == mv
        onehot = jnp.where(mask, 1.0, 0.0)
        hl = jnp.dot(iota_rows, onehot, preferred_element_type=jnp.float32)
        mis.append(hl[0:1] * 32.0 + hl[1:2])
        d2k = jnp.maximum(mv, 1e-10)
        invs.append(1.0 / (jnp.sqrt(d2k) + 1e-8))
        if k < 2:
            qw = jnp.where(mask, jnp.float32(jnp.inf), qw)
    s = invs[0] + invs[1] + invs[2]
    idx_ref[0] = jnp.concatenate(
        [jnp.clip(mi, 0.0, m_total - 1).astype(jnp.int32) for mi in mis],
        axis=0)
    w_ref[0] = jnp.concatenate([v / s for v in invs], axis=0)


def _mlp_body(x_ref, w1_ref, b1_ref, w2_ref, b2_ref, o_ref):
    h = jnp.maximum(
        jnp.dot(w1_ref[...], x_ref[...], preferred_element_type=jnp.float32)
        + b1_ref[...], 0.0)
    o_ref[...] = jnp.maximum(
        jnp.dot(w2_ref[...], h, preferred_element_type=jnp.float32)
        + b2_ref[...], 0.0)


def kernel(points_coords, centers_coords, centers_features, condition,
           W1, b1, W2, b2):
    B, _, N = points_coords.shape
    M = centers_coords.shape[2]
    CIN = centers_features.shape[1]
    C1 = W1.shape[0]
    C2 = W2.shape[0]
    NT = N // TILE_N
    PTS = N // NWORKERS
    TPW = PTS // TILE_N
    GROUPS = PTS // LANES

    knn = pl.pallas_call(
        functools.partial(_knn_body, m_total=M),
        grid=(NT,),
        in_specs=[
            pl.BlockSpec((M, 3), lambda t: (0, 0)),
            pl.BlockSpec((3, TILE_N), lambda t: (0, t)),
        ],
        out_specs=[
            pl.BlockSpec((1, 3, TILE_N), lambda t: (t, 0, 0)),
            pl.BlockSpec((1, 3, TILE_N), lambda t: (t, 0, 0)),
        ],
        out_shape=[
            jax.ShapeDtypeStruct((NT, 3, TILE_N), jnp.int32),
            jax.ShapeDtypeStruct((NT, 3, TILE_N), jnp.float32),
        ],
    )

    mesh = plsc.VectorSubcoreMesh(core_axis_name="c", subcore_axis_name="s")

    @functools.partial(
        pl.kernel, mesh=mesh,
        compiler_params=pltpu.CompilerParams(
            use_tc_tiling_on_sc=False, needs_layout_passes=False),
        out_type=jax.ShapeDtypeStruct((CIN, N), jnp.float32),
        scratch_types=[
            pltpu.VMEM((CIN, M), jnp.float32),
            pltpu.VMEM((TPW, 3, TILE_N), jnp.int32),
            pltpu.VMEM((TPW, 3, TILE_N), jnp.float32),
            pltpu.VMEM((CIN, PTS), jnp.float32),
        ],
    )
    def sc_interp(feat_hbm, idx_hbm, w_hbm, out_hbm,
                  table_v, idx_v, w_v, out_v):
        wid = lax.axis_index("s") * 2 + lax.axis_index("c")
        pltpu.sync_copy(feat_hbm, table_v)
        pltpu.sync_copy(idx_hbm.at[pl.ds(wid * TPW, TPW)], idx_v)
        pltpu.sync_copy(w_hbm.at[pl.ds(wid * TPW, TPW)], w_v)
        for t in range(TPW):
            @plsc.parallel_loop(0, TILE_N // LANES, unroll=4)
            def body(g, t=t):
                i0 = idx_v[t, 0, pl.ds(g * LANES, LANES)]
                i1 = idx_v[t, 1, pl.ds(g * LANES, LANES)]
                i2 = idx_v[t, 2, pl.ds(g * LANES, LANES)]
                w0 = w_v[t, 0, pl.ds(g * LANES, LANES)]
                w1_ = w_v[t, 1, pl.ds(g * LANES, LANES)]
                w2_ = w_v[t, 2, pl.ds(g * LANES, LANES)]
                off = t * TILE_N + g * LANES
                for c in range(CIN):
                    cvec = jnp.full((LANES,), c, jnp.int32)
                    acc = (w0 * plsc.load_gather(table_v, [cvec, i0])
                           + w1_ * plsc.load_gather(table_v, [cvec, i1])
                           + w2_ * plsc.load_gather(table_v, [cvec, i2]))
                    out_v[c, pl.ds(off, LANES)] = acc
        pltpu.sync_copy(out_v, out_hbm.at[:, pl.ds(wid * PTS, PTS)])

    mlp = pl.pallas_call(
        _mlp_body,
        grid=(N // TILE_MLP,),
        in_specs=[
            pl.BlockSpec((CIN, TILE_MLP), lambda t: (0, t)),
            pl.BlockSpec((C1, CIN), lambda t: (0, 0)),
            pl.BlockSpec((C1, 1), lambda t: (0, 0)),
            pl.BlockSpec((C2, C1), lambda t: (0, 0)),
            pl.BlockSpec((C2, 1), lambda t: (0, 0)),
        ],
        out_specs=pl.BlockSpec((C2, TILE_MLP), lambda t: (0, t)),
        out_shape=jax.ShapeDtypeStruct((C2, N), jnp.float32),
    )

    ct = jnp.swapaxes(centers_coords, 1, 2)
    b1c = b1.reshape(C1, 1)
    b2c = b2.reshape(C2, 1)
    res = [knn(ct[b], points_coords[b]) for b in range(B)]
    interps = [sc_interp(centers_features[b], res[b][0], res[b][1])
               for b in range(B)]
    outs = [mlp(x, W1, b1c, W2, b2c) for x in interps]
    return (jnp.stack(outs), points_coords, condition)


# speedup vs baseline: 47.3522x; 1.0217x over previous
import functools

import jax
import jax.numpy as jnp
from jax import lax
from jax.experimental import pallas as pl
from jax.experimental.pallas import tpu as pltpu
from jax.experimental.pallas import tpu_sc as plsc

TILE_N = 512
TILE_MLP = 1024
LANES = 16
NWORKERS = 32


def _knn_body(ct_ref, p_ref, idx_ref, w_ref, *, m_total):
    ct = ct_ref[...]
    p = p_ref[...]
    pn2 = jnp.sum(p * p, axis=0, keepdims=True)
    cm2 = jnp.sum(ct * ct, axis=1, keepdims=True)
    q = (cm2 - 2.0 * jnp.dot(ct, p, preferred_element_type=jnp.float32)
         + pn2)
    ii = lax.broadcasted_iota(jnp.int32, (8, q.shape[0]), 1)
    rr = lax.broadcasted_iota(jnp.int32, (8, q.shape[0]), 0)
    hi = jnp.where(rr == 0, ii >> 5, 0).astype(jnp.float32)
    lo = jnp.where(rr == 1, ii & 31, 0).astype(jnp.float32)
    iota_rows = hi + lo
    mis, invs = [], []
    qw = q
    for k in range(3):
        mv = jnp.min(qw, axis=0, keepdims=True)
        mask = qw == mv
        onehot = jnp.where(mask, 1.0, 0.0)
        hl = jnp.dot(iota_rows, onehot, preferred_element_type=jnp.float32)
        mis.append(hl[0:1] * 32.0 + hl[1:2])
        d2k = jnp.maximum(mv, 1e-10)
        invs.append(1.0 / (jnp.sqrt(d2k) + 1e-8))
        if k < 2:
            qw = jnp.where(mask, jnp.float32(jnp.inf), qw)
    s = invs[0] + invs[1] + invs[2]
    zpad = jnp.zeros((5, q.shape[1]), jnp.float32)
    idx_ref[0] = jnp.concatenate(
        [jnp.clip(mi, 0.0, m_total - 1) for mi in mis] + [zpad],
        axis=0).astype(jnp.int32)
    w_ref[0] = jnp.concatenate([v / s for v in invs] + [zpad], axis=0)


def _mlp_body(x_ref, w1_ref, b1_ref, w2_ref, b2_ref, o_ref):
    h = jnp.maximum(
        jnp.dot(w1_ref[...], x_ref[0], preferred_element_type=jnp.float32)
        + b1_ref[...], 0.0)
    o_ref[0] = jnp.maximum(
        jnp.dot(w2_ref[...], h, preferred_element_type=jnp.float32)
        + b2_ref[...], 0.0)


def kernel(points_coords, centers_coords, centers_features, condition,
           W1, b1, W2, b2):
    B, _, N = points_coords.shape
    M = centers_coords.shape[2]
    CIN = centers_features.shape[1]
    C1 = W1.shape[0]
    C2 = W2.shape[0]
    NT = N // TILE_N
    PTS = N // NWORKERS
    TPW = PTS // TILE_N
    GROUPS = PTS // LANES

    knn = pl.pallas_call(
        functools.partial(_knn_body, m_total=M),
        grid=(NT,),
        in_specs=[
            pl.BlockSpec((M, 3), lambda t: (0, 0)),
            pl.BlockSpec((3, TILE_N), lambda t: (0, t)),
        ],
        out_specs=[
            pl.BlockSpec((1, 8, TILE_N), lambda t: (t, 0, 0)),
            pl.BlockSpec((1, 8, TILE_N), lambda t: (t, 0, 0)),
        ],
        out_shape=[
            jax.ShapeDtypeStruct((NT, 8, TILE_N), jnp.int32),
            jax.ShapeDtypeStruct((NT, 8, TILE_N), jnp.float32),
        ],
    )

    mesh = plsc.VectorSubcoreMesh(core_axis_name="c", subcore_axis_name="s")

    @functools.partial(
        pl.kernel, mesh=mesh,
        compiler_params=pltpu.CompilerParams(
            use_tc_tiling_on_sc=False, needs_layout_passes=False),
        out_type=jax.ShapeDtypeStruct((CIN, N), jnp.float32),
        scratch_types=[
            pltpu.VMEM((CIN, M), jnp.float32),
            pltpu.VMEM((TPW, 8, TILE_N), jnp.int32),
            pltpu.VMEM((TPW, 8, TILE_N), jnp.float32),
            pltpu.VMEM((CIN, PTS), jnp.float32),
        ],
    )
    def sc_interp(feat_hbm, idx_hbm, w_hbm, out_hbm,
                  table_v, idx_v, w_v, out_v):
        wid = lax.axis_index("s") * 2 + lax.axis_index("c")
        pltpu.sync_copy(feat_hbm, table_v)
        pltpu.sync_copy(idx_hbm.at[pl.ds(wid * TPW, TPW)], idx_v)
        pltpu.sync_copy(w_hbm.at[pl.ds(wid * TPW, TPW)], w_v)
        for t in range(TPW):
            @plsc.parallel_loop(0, TILE_N // LANES, unroll=4)
            def body(g, t=t):
                i0 = idx_v[t, 0, pl.ds(g * LANES, LANES)]
                i1 = idx_v[t, 1, pl.ds(g * LANES, LANES)]
                i2 = idx_v[t, 2, pl.ds(g * LANES, LANES)]
                w0 = w_v[t, 0, pl.ds(g * LANES, LANES)]
                w1_ = w_v[t, 1, pl.ds(g * LANES, LANES)]
                w2_ = w_v[t, 2, pl.ds(g * LANES, LANES)]
                off = t * TILE_N + g * LANES
                for c in range(CIN):
                    cvec = jnp.full((LANES,), c, jnp.int32)
                    acc = (w0 * plsc.load_gather(table_v, [cvec, i0])
                           + w1_ * plsc.load_gather(table_v, [cvec, i1])
                           + w2_ * plsc.load_gather(table_v, [cvec, i2]))
                    out_v[c, pl.ds(off, LANES)] = acc
        pltpu.sync_copy(out_v, out_hbm.at[:, pl.ds(wid * PTS, PTS)])

    mlp = pl.pallas_call(
        _mlp_body,
        grid=(B, N // TILE_MLP),
        in_specs=[
            pl.BlockSpec((1, CIN, TILE_MLP), lambda b, t: (b, 0, t)),
            pl.BlockSpec((C1, CIN), lambda b, t: (0, 0)),
            pl.BlockSpec((C1, 1), lambda b, t: (0, 0)),
            pl.BlockSpec((C2, C1), lambda b, t: (0, 0)),
            pl.BlockSpec((C2, 1), lambda b, t: (0, 0)),
        ],
        out_specs=pl.BlockSpec((1, C2, TILE_MLP), lambda b, t: (b, 0, t)),
        out_shape=jax.ShapeDtypeStruct((B, C2, N), jnp.float32),
    )

    ct = jnp.swapaxes(centers_coords, 1, 2)
    b1c = b1.reshape(C1, 1)
    b2c = b2.reshape(C2, 1)
    res = [knn(ct[b], points_coords[b]) for b in range(B)]
    interps = [sc_interp(centers_features[b], res[b][0], res[b][1])
               for b in range(B)]
    out = mlp(jnp.stack(interps), W1, b1c, W2, b2c)
    return (out, points_coords, condition)
